# baseline (speedup 1.0000x reference)
import jax
import jax.numpy as jnp
import numpy as np
from jax.experimental import pallas as pl
from jax.experimental.pallas import tpu as pltpu

_EPSILON = float(np.finfo(np.float32).tiny)
_K = 32
_TAU = 0.1

_ROW_BLOCK = 16


def _gumbel_topk_kernel(x_ref, g_ref, out_ref):
    s = x_ref[...] + g_ref[...]
    t = s / _TAU
    m = jnp.max(t, axis=1, keepdims=True)
    e = jnp.exp(t - m)
    d = jnp.sum(e, axis=1, keepdims=True)
    khot = e * 0.0

    def soft_body(_, carry):
        s, khot, e, d = carry
        onehot = e / d
        khot = khot + onehot
        khot_mask = jnp.maximum(1.0 - onehot, _EPSILON)
        s = s + jnp.log(khot_mask)
        t = s / _TAU
        m = jnp.max(t, axis=1, keepdims=True)
        e = jnp.exp(t - m)
        d = jnp.sum(e, axis=1, keepdims=True)
        return (s, khot, e, d)

    s, khot, e, d = jax.lax.fori_loop(
        0, _K - 1, soft_body, (s, khot, e, d), unroll=False
    )
    khot = khot + e / d

    iota = jax.lax.broadcasted_iota(jnp.int32, khot.shape, 1)
    big = jnp.int32(np.iinfo(np.int32).max)

    def topk_body(_, carry):
        w, hard = carry
        m = jnp.max(w, axis=1, keepdims=True)
        idx = jnp.min(jnp.where(w == m, iota, big), axis=1, keepdims=True)
        sel = iota == idx
        hard = jnp.where(sel, 1.0, hard)
        w = jnp.where(sel, -jnp.inf, w)
        return (w, hard)

    _, hard = jax.lax.fori_loop(
        0, _K, topk_body, (khot, khot * 0.0), unroll=False
    )

    out_ref[...] = (hard - khot) + khot


def kernel(scores):
    bsz, nmax, _, ensemble = scores.shape
    rows = bsz * ensemble
    cols = nmax * nmax
    x = jnp.transpose(scores, (0, 3, 1, 2)).reshape(rows, cols)
    g = jax.random.gumbel(jax.random.key(42), x.shape, dtype=x.dtype)

    res = pl.pallas_call(
        _gumbel_topk_kernel,
        grid=(rows // _ROW_BLOCK,),
        in_specs=[
            pl.BlockSpec((_ROW_BLOCK, cols), lambda i: (i, 0)),
            pl.BlockSpec((_ROW_BLOCK, cols), lambda i: (i, 0)),
        ],
        out_specs=pl.BlockSpec((_ROW_BLOCK, cols), lambda i: (i, 0)),
        out_shape=jax.ShapeDtypeStruct((rows, cols), x.dtype),
        compiler_params=pltpu.CompilerParams(
            dimension_semantics=("parallel",),
        ),
    )(x, g)

    res = res.reshape(bsz, ensemble, nmax, nmax)
    return jnp.transpose(res, (0, 2, 3, 1))


# speedup vs baseline: 1.0420x; 1.0420x over previous
import jax
import jax.numpy as jnp
import numpy as np
from jax.experimental import pallas as pl
from jax.experimental.pallas import tpu as pltpu

_EPSILON = float(np.finfo(np.float32).tiny)
_K = 32
_TAU = 0.1

_ROW_BLOCK = 16


def _gumbel_topk_kernel(x_ref, g_ref, out_ref):
    s = x_ref[...] + g_ref[...]
    khot = jnp.zeros_like(s)
    onehot = jnp.zeros_like(s)
    inv_tau = jnp.float32(1.0) / jnp.float32(_TAU)

    def soft_body(_, carry):
        s, khot, onehot = carry
        khot_mask = jnp.maximum(1.0 - onehot, _EPSILON)
        s = s + jnp.log(khot_mask)
        t = s * inv_tau
        m = jnp.max(t, axis=1, keepdims=True)
        e = jnp.exp(t - m)
        d = jnp.sum(e, axis=1, keepdims=True)
        onehot = e * (1.0 / d)
        khot = khot + onehot
        return (s, khot, onehot)

    s, khot, onehot = jax.lax.fori_loop(
        0, _K, soft_body, (s, khot, onehot), unroll=False
    )

    iota = jax.lax.broadcasted_iota(jnp.int32, khot.shape, 1)
    big = jnp.int32(np.iinfo(np.int32).max)

    def topk_body(_, carry):
        w, hard = carry
        m = jnp.max(w, axis=1, keepdims=True)
        idx = jnp.min(jnp.where(w == m, iota, big), axis=1, keepdims=True)
        sel = iota == idx
        hard = jnp.where(sel, 1.0, hard)
        w = jnp.where(sel, -jnp.inf, w)
        return (w, hard)

    _, hard = jax.lax.fori_loop(
        0, _K, topk_body, (khot, khot * 0.0), unroll=False
    )

    out_ref[...] = (hard - khot) + khot


def kernel(scores):
    bsz, nmax, _, ensemble = scores.shape
    rows = bsz * ensemble
    cols = nmax * nmax
    x = jnp.transpose(scores, (0, 3, 1, 2)).reshape(rows, cols)
    g = jax.random.gumbel(jax.random.key(42), x.shape, dtype=x.dtype)

    res = pl.pallas_call(
        _gumbel_topk_kernel,
        grid=(rows // _ROW_BLOCK,),
        in_specs=[
            pl.BlockSpec((_ROW_BLOCK, cols), lambda i: (i, 0)),
            pl.BlockSpec((_ROW_BLOCK, cols), lambda i: (i, 0)),
        ],
        out_specs=pl.BlockSpec((_ROW_BLOCK, cols), lambda i: (i, 0)),
        out_shape=jax.ShapeDtypeStruct((rows, cols), x.dtype),
        compiler_params=pltpu.CompilerParams(
            dimension_semantics=("parallel",),
        ),
    )(x, g)

    res = res.reshape(bsz, ensemble, nmax, nmax)
    return jnp.transpose(res, (0, 2, 3, 1))


# speedup vs baseline: 1.5334x; 1.4716x over previous
import jax
import jax.numpy as jnp
import numpy as np
from jax.experimental import pallas as pl
from jax.experimental.pallas import tpu as pltpu

_EPSILON = float(np.finfo(np.float32).tiny)
_K = 32
_TAU = 0.1

_ROW_BLOCK = 16
_CHUNK = 1024


def _gumbel_topk_kernel(x_ref, g_ref, out_ref, s_scr, khot_scr, e_scr):
    rows, cols = x_ref.shape
    ncheck = cols // _CHUNK
    chunks = [slice(c * _CHUNK, (c + 1) * _CHUNK) for c in range(ncheck)]
    inv_tau = jnp.float32(1.0) / jnp.float32(_TAU)

    def chunk_iota(c):
        it = jax.lax.broadcasted_iota(jnp.int32, (rows, _CHUNK), 1)
        return it + jnp.int32(c * _CHUNK)

    macc = None
    for sl in chunks:
        sv = x_ref[:, sl] + g_ref[:, sl]
        s_scr[:, sl] = sv
        khot_scr[:, sl] = sv * 0.0
        t = sv * inv_tau
        macc = t if macc is None else jnp.maximum(macc, t)
    m = jnp.max(macc, axis=1, keepdims=True)
    for sl in chunks:
        t = s_scr[:, sl] * inv_tau
        e_scr[:, sl] = jnp.exp(t - m)
    d = jnp.sum(e_scr[...], axis=1, keepdims=True)

    def soft_body(_, d):
        invd = 1.0 / d
        macc = None
        for sl in chunks:
            onehot = e_scr[:, sl] * invd
            khot_scr[:, sl] = khot_scr[:, sl] + onehot
            khot_mask = jnp.maximum(1.0 - onehot, _EPSILON)
            sv = s_scr[:, sl] + jnp.log(khot_mask)
            s_scr[:, sl] = sv
            t = sv * inv_tau
            macc = t if macc is None else jnp.maximum(macc, t)
        m = jnp.max(macc, axis=1, keepdims=True)
        for sl in chunks:
            t = s_scr[:, sl] * inv_tau
            e_scr[:, sl] = jnp.exp(t - m)
        return jnp.sum(e_scr[...], axis=1, keepdims=True)

    d = jax.lax.fori_loop(0, _K - 1, soft_body, d, unroll=False)

    invd = 1.0 / d
    for sl in chunks:
        khot_scr[:, sl] = khot_scr[:, sl] + e_scr[:, sl] * invd

    big = jnp.int32(np.iinfo(np.int32).max)
    neg_inf = jnp.float32(-np.inf)
    m_prev = d * 0.0 + jnp.float32(np.inf)
    idx_prev = (d * 0.0 - 1.0).astype(jnp.int32)

    def ext_body(_, carry):
        m_prev, idx_prev = carry
        macc = None
        for c, sl in enumerate(chunks):
            v = khot_scr[:, sl]
            it = chunk_iota(c)
            valid = (v < m_prev) | ((v == m_prev) & (it > idx_prev))
            t = jnp.where(valid, v, neg_inf)
            macc = t if macc is None else jnp.maximum(macc, t)
        m_new = jnp.max(macc, axis=1, keepdims=True)
        iacc = None
        for c, sl in enumerate(chunks):
            v = khot_scr[:, sl]
            it = chunk_iota(c)
            cand = (v == m_new) & ((v < m_prev) | ((v == m_prev) & (it > idx_prev)))
            ii = jnp.where(cand, it, big)
            iacc = ii if iacc is None else jnp.minimum(iacc, ii)
        idx_new = jnp.min(iacc, axis=1, keepdims=True)
        return m_new, idx_new

    theta, idx_last = jax.lax.fori_loop(
        0, _K, ext_body, (m_prev, idx_prev), unroll=False
    )

    for c, sl in enumerate(chunks):
        v = khot_scr[:, sl]
        it = chunk_iota(c)
        sel = (v > theta) | ((v == theta) & (it <= idx_last))
        hard = jnp.where(sel, jnp.float32(1.0), jnp.float32(0.0))
        out_ref[:, sl] = (hard - v) + v


def kernel(scores):
    bsz, nmax, _, ensemble = scores.shape
    rows = bsz * ensemble
    cols = nmax * nmax
    x = jnp.transpose(scores, (0, 3, 1, 2)).reshape(rows, cols)
    g = jax.random.gumbel(jax.random.key(42), x.shape, dtype=x.dtype)

    res = pl.pallas_call(
        _gumbel_topk_kernel,
        grid=(rows // _ROW_BLOCK,),
        in_specs=[
            pl.BlockSpec((_ROW_BLOCK, cols), lambda i: (i, 0)),
            pl.BlockSpec((_ROW_BLOCK, cols), lambda i: (i, 0)),
        ],
        out_specs=pl.BlockSpec((_ROW_BLOCK, cols), lambda i: (i, 0)),
        out_shape=jax.ShapeDtypeStruct((rows, cols), x.dtype),
        scratch_shapes=[
            pltpu.VMEM((_ROW_BLOCK, cols), jnp.float32),
            pltpu.VMEM((_ROW_BLOCK, cols), jnp.float32),
            pltpu.VMEM((_ROW_BLOCK, cols), jnp.float32),
        ],
        compiler_params=pltpu.CompilerParams(
            dimension_semantics=("parallel",),
        ),
    )(x, g)

    res = res.reshape(bsz, ensemble, nmax, nmax)
    return jnp.transpose(res, (0, 2, 3, 1))


# speedup vs baseline: 1.6736x; 1.0914x over previous
import jax
import jax.numpy as jnp
import numpy as np
from jax.experimental import pallas as pl
from jax.experimental.pallas import tpu as pltpu

_EPSILON = float(np.finfo(np.float32).tiny)
_K = 32
_TAU = 0.1

_ROW_BLOCK = 32
_CHUNK = 1024


def _gumbel_topk_kernel(x_ref, g_ref, out_ref, s_scr, khot_scr, e_scr):
    rows, cols = x_ref.shape
    ncheck = cols // _CHUNK
    chunks = [slice(c * _CHUNK, (c + 1) * _CHUNK) for c in range(ncheck)]
    inv_tau = jnp.float32(1.0) / jnp.float32(_TAU)

    def chunk_iota(c):
        it = jax.lax.broadcasted_iota(jnp.int32, (rows, _CHUNK), 1)
        return it + jnp.int32(c * _CHUNK)

    macc = None
    for sl in chunks:
        sv = x_ref[:, sl] + g_ref[:, sl]
        s_scr[:, sl] = sv
        khot_scr[:, sl] = sv * 0.0
        t = sv * inv_tau
        macc = t if macc is None else jnp.maximum(macc, t)
    m = jnp.max(macc, axis=1, keepdims=True)
    for sl in chunks:
        t = s_scr[:, sl] * inv_tau
        e_scr[:, sl] = jnp.exp(t - m)
    d = jnp.sum(e_scr[...], axis=1, keepdims=True)

    def soft_body(_, d):
        invd = 1.0 / d
        macc = None
        for sl in chunks:
            onehot = e_scr[:, sl] * invd
            khot_scr[:, sl] = khot_scr[:, sl] + onehot
            khot_mask = jnp.maximum(1.0 - onehot, _EPSILON)
            sv = s_scr[:, sl] + jnp.log(khot_mask)
            s_scr[:, sl] = sv
            t = sv * inv_tau
            macc = t if macc is None else jnp.maximum(macc, t)
        m = jnp.max(macc, axis=1, keepdims=True)
        for sl in chunks:
            t = s_scr[:, sl] * inv_tau
            e_scr[:, sl] = jnp.exp(t - m)
        return jnp.sum(e_scr[...], axis=1, keepdims=True)

    d = jax.lax.fori_loop(0, _K - 1, soft_body, d, unroll=False)

    invd = 1.0 / d
    for sl in chunks:
        khot_scr[:, sl] = khot_scr[:, sl] + e_scr[:, sl] * invd

    big = jnp.int32(np.iinfo(np.int32).max)
    neg_inf = jnp.float32(-np.inf)
    m_prev = d * 0.0 + jnp.float32(np.inf)
    idx_prev = (d * 0.0 - 1.0).astype(jnp.int32)

    def ext_body(_, carry):
        m_prev, idx_prev = carry
        macc = None
        for c, sl in enumerate(chunks):
            v = khot_scr[:, sl]
            it = chunk_iota(c)
            valid = (v < m_prev) | ((v == m_prev) & (it > idx_prev))
            t = jnp.where(valid, v, neg_inf)
            macc = t if macc is None else jnp.maximum(macc, t)
        m_new = jnp.max(macc, axis=1, keepdims=True)
        iacc = None
        for c, sl in enumerate(chunks):
            v = khot_scr[:, sl]
            it = chunk_iota(c)
            cand = (v == m_new) & ((v < m_prev) | ((v == m_prev) & (it > idx_prev)))
            ii = jnp.where(cand, it, big)
            iacc = ii if iacc is None else jnp.minimum(iacc, ii)
        idx_new = jnp.min(iacc, axis=1, keepdims=True)
        return m_new, idx_new

    theta, idx_last = jax.lax.fori_loop(
        0, _K, ext_body, (m_prev, idx_prev), unroll=False
    )

    for c, sl in enumerate(chunks):
        v = khot_scr[:, sl]
        it = chunk_iota(c)
        sel = (v > theta) | ((v == theta) & (it <= idx_last))
        hard = jnp.where(sel, jnp.float32(1.0), jnp.float32(0.0))
        out_ref[:, sl] = (hard - v) + v


def kernel(scores):
    bsz, nmax, _, ensemble = scores.shape
    rows = bsz * ensemble
    cols = nmax * nmax
    x = jnp.transpose(scores, (0, 3, 1, 2)).reshape(rows, cols)
    g = jax.random.gumbel(jax.random.key(42), x.shape, dtype=x.dtype)

    res = pl.pallas_call(
        _gumbel_topk_kernel,
        grid=(rows // _ROW_BLOCK,),
        in_specs=[
            pl.BlockSpec((_ROW_BLOCK, cols), lambda i: (i, 0)),
            pl.BlockSpec((_ROW_BLOCK, cols), lambda i: (i, 0)),
        ],
        out_specs=pl.BlockSpec((_ROW_BLOCK, cols), lambda i: (i, 0)),
        out_shape=jax.ShapeDtypeStruct((rows, cols), x.dtype),
        scratch_shapes=[
            pltpu.VMEM((_ROW_BLOCK, cols), jnp.float32),
            pltpu.VMEM((_ROW_BLOCK, cols), jnp.float32),
            pltpu.VMEM((_ROW_BLOCK, cols), jnp.float32),
        ],
        compiler_params=pltpu.CompilerParams(
            dimension_semantics=("parallel",),
        ),
    )(x, g)

    res = res.reshape(bsz, ensemble, nmax, nmax)
    return jnp.transpose(res, (0, 2, 3, 1))


# speedup vs baseline: 1.7580x; 1.0504x over previous
import jax
import jax.numpy as jnp
import numpy as np
from jax.experimental import pallas as pl
from jax.experimental.pallas import tpu as pltpu

_EPSILON = float(np.finfo(np.float32).tiny)
_K = 32
_TAU = 0.1

_ROW_BLOCK = 64
_CHUNK = 1024


def _gumbel_topk_kernel(x_ref, g_ref, out_ref, s_scr, khot_scr, e_scr):
    rows, cols = x_ref.shape
    ncheck = cols // _CHUNK
    chunks = [slice(c * _CHUNK, (c + 1) * _CHUNK) for c in range(ncheck)]
    inv_tau = jnp.float32(1.0) / jnp.float32(_TAU)

    def chunk_iota(c):
        it = jax.lax.broadcasted_iota(jnp.int32, (rows, _CHUNK), 1)
        return it + jnp.int32(c * _CHUNK)

    macc = None
    for sl in chunks:
        sv = x_ref[:, sl] + g_ref[:, sl]
        s_scr[:, sl] = sv
        khot_scr[:, sl] = sv * 0.0
        t = sv * inv_tau
        macc = t if macc is None else jnp.maximum(macc, t)
    m = jnp.max(macc, axis=1, keepdims=True)
    for sl in chunks:
        t = s_scr[:, sl] * inv_tau
        e_scr[:, sl] = jnp.exp(t - m)
    d = jnp.sum(e_scr[...], axis=1, keepdims=True)

    def soft_body(_, d):
        invd = 1.0 / d
        macc = None
        for sl in chunks:
            onehot = e_scr[:, sl] * invd
            khot_scr[:, sl] = khot_scr[:, sl] + onehot
            khot_mask = jnp.maximum(1.0 - onehot, _EPSILON)
            sv = s_scr[:, sl] + jnp.log(khot_mask)
            s_scr[:, sl] = sv
            t = sv * inv_tau
            macc = t if macc is None else jnp.maximum(macc, t)
        m = jnp.max(macc, axis=1, keepdims=True)
        for sl in chunks:
            t = s_scr[:, sl] * inv_tau
            e_scr[:, sl] = jnp.exp(t - m)
        return jnp.sum(e_scr[...], axis=1, keepdims=True)

    d = jax.lax.fori_loop(0, _K - 1, soft_body, d, unroll=False)

    invd = 1.0 / d
    for sl in chunks:
        khot_scr[:, sl] = khot_scr[:, sl] + e_scr[:, sl] * invd

    big = jnp.int32(np.iinfo(np.int32).max)
    neg_inf = jnp.float32(-np.inf)
    m_prev = d * 0.0 + jnp.float32(np.inf)
    idx_prev = (d * 0.0 - 1.0).astype(jnp.int32)

    def ext_body(_, carry):
        m_prev, idx_prev = carry
        macc = None
        for c, sl in enumerate(chunks):
            v = khot_scr[:, sl]
            it = chunk_iota(c)
            valid = (v < m_prev) | ((v == m_prev) & (it > idx_prev))
            t = jnp.where(valid, v, neg_inf)
            macc = t if macc is None else jnp.maximum(macc, t)
        m_new = jnp.max(macc, axis=1, keepdims=True)
        iacc = None
        for c, sl in enumerate(chunks):
            v = khot_scr[:, sl]
            it = chunk_iota(c)
            cand = (v == m_new) & ((v < m_prev) | ((v == m_prev) & (it > idx_prev)))
            ii = jnp.where(cand, it, big)
            iacc = ii if iacc is None else jnp.minimum(iacc, ii)
        idx_new = jnp.min(iacc, axis=1, keepdims=True)
        return m_new, idx_new

    theta, idx_last = jax.lax.fori_loop(
        0, _K, ext_body, (m_prev, idx_prev), unroll=False
    )

    for c, sl in enumerate(chunks):
        v = khot_scr[:, sl]
        it = chunk_iota(c)
        sel = (v > theta) | ((v == theta) & (it <= idx_last))
        hard = jnp.where(sel, jnp.float32(1.0), jnp.float32(0.0))
        out_ref[:, sl] = (hard - v) + v


def kernel(scores):
    bsz, nmax, _, ensemble = scores.shape
    rows = bsz * ensemble
    cols = nmax * nmax
    x = jnp.transpose(scores, (0, 3, 1, 2)).reshape(rows, cols)
    g = jax.random.gumbel(jax.random.key(42), x.shape, dtype=x.dtype)

    res = pl.pallas_call(
        _gumbel_topk_kernel,
        grid=(rows // _ROW_BLOCK,),
        in_specs=[
            pl.BlockSpec((_ROW_BLOCK, cols), lambda i: (i, 0)),
            pl.BlockSpec((_ROW_BLOCK, cols), lambda i: (i, 0)),
        ],
        out_specs=pl.BlockSpec((_ROW_BLOCK, cols), lambda i: (i, 0)),
        out_shape=jax.ShapeDtypeStruct((rows, cols), x.dtype),
        scratch_shapes=[
            pltpu.VMEM((_ROW_BLOCK, cols), jnp.float32),
            pltpu.VMEM((_ROW_BLOCK, cols), jnp.float32),
            pltpu.VMEM((_ROW_BLOCK, cols), jnp.float32),
        ],
        compiler_params=pltpu.CompilerParams(
            dimension_semantics=("parallel",),
        ),
    )(x, g)

    res = res.reshape(bsz, ensemble, nmax, nmax)
    return jnp.transpose(res, (0, 2, 3, 1))


# speedup vs baseline: 2.5412x; 1.4456x over previous
import jax
import jax.numpy as jnp
import numpy as np
from jax.experimental import pallas as pl
from jax.experimental.pallas import tpu as pltpu

_EPSILON = float(np.finfo(np.float32).tiny)
_K = 32
_TAU = 0.1

_ROW_BLOCK = 64
_CHUNK = 1024


def _gumbel_topk_kernel(x_ref, g_ref, out_ref, s_scr, khot_scr, e_scr):
    rows, cols = x_ref.shape
    ncheck = cols // _CHUNK
    chunks = [slice(c * _CHUNK, (c + 1) * _CHUNK) for c in range(ncheck)]
    inv_tau = jnp.float32(1.0) / jnp.float32(_TAU)

    def chunk_iota(c):
        it = jax.lax.broadcasted_iota(jnp.int32, (rows, _CHUNK), 1)
        return it + jnp.int32(c * _CHUNK)

    macc = None
    for sl in chunks:
        sv = x_ref[:, sl] + g_ref[:, sl]
        s_scr[:, sl] = sv
        khot_scr[:, sl] = jnp.abs(sv * 0.0)
        t = sv * inv_tau
        macc = t if macc is None else jnp.maximum(macc, t)
    m = jnp.max(macc, axis=1, keepdims=True)
    for sl in chunks:
        t = s_scr[:, sl] * inv_tau
        e_scr[:, sl] = jnp.exp(t - m)
    d = jnp.sum(e_scr[...], axis=1, keepdims=True)

    def soft_body(_, d):
        invd = 1.0 / d
        macc = None
        for sl in chunks:
            onehot = e_scr[:, sl] * invd
            khot_scr[:, sl] = khot_scr[:, sl] + onehot
            khot_mask = jnp.maximum(1.0 - onehot, _EPSILON)
            sv = s_scr[:, sl] + jnp.log(khot_mask)
            s_scr[:, sl] = sv
            t = sv * inv_tau
            macc = t if macc is None else jnp.maximum(macc, t)
        m = jnp.max(macc, axis=1, keepdims=True)
        for sl in chunks:
            t = s_scr[:, sl] * inv_tau
            e_scr[:, sl] = jnp.exp(t - m)
        return jnp.sum(e_scr[...], axis=1, keepdims=True)

    d = jax.lax.fori_loop(0, _K - 1, soft_body, d, unroll=False)

    invd = 1.0 / d
    for sl in chunks:
        khot_scr[:, sl] = khot_scr[:, sl] + e_scr[:, sl] * invd

    def khot_bits(sl):
        return jax.lax.bitcast_convert_type(khot_scr[:, sl], jnp.int32)

    def count_rows(acc):
        return jnp.sum(acc, axis=1, keepdims=True)

    lo0 = (d * 0.0).astype(jnp.int32)
    hi0 = lo0 + jnp.int32(0x7F800000)

    def vsearch_body(_, carry):
        lo, hi = carry
        mid = lo + jax.lax.shift_right_logical(hi - lo, 1)
        acc = None
        for sl in chunks:
            one = jnp.where(khot_bits(sl) >= mid, 1.0, 0.0)
            acc = one if acc is None else acc + one
        take = count_rows(acc) >= 32.0
        lo = jnp.where(take, mid, lo)
        hi = jnp.where(take, hi, mid)
        return lo, hi

    theta_b, _ = jax.lax.fori_loop(0, 31, vsearch_body, (lo0, hi0), unroll=False)
    theta = jax.lax.bitcast_convert_type(theta_b, jnp.float32)

    accg = None
    for sl in chunks:
        one = jnp.where(khot_bits(sl) >= theta_b + 1, 1.0, 0.0)
        accg = one if accg is None else accg + one
    r = 32.0 - count_rows(accg)

    ilo0 = lo0 - 1
    ihi0 = lo0 + jnp.int32(cols - 1)

    def isearch_body(_, carry):
        lo, hi = carry
        mid = lo + jax.lax.shift_right_logical(hi - lo, 1)
        acc = None
        for c, sl in enumerate(chunks):
            hit = (khot_bits(sl) == theta_b) & (chunk_iota(c) <= mid)
            one = jnp.where(hit, 1.0, 0.0)
            acc = one if acc is None else acc + one
        take = count_rows(acc) >= r
        lo = jnp.where(take, lo, mid)
        hi = jnp.where(take, mid, hi)
        return lo, hi

    _, idx_cut = jax.lax.fori_loop(0, 14, isearch_body, (ilo0, ihi0), unroll=False)

    for c, sl in enumerate(chunks):
        v = khot_scr[:, sl]
        it = chunk_iota(c)
        sel = (v > theta) | ((v == theta) & (it <= idx_cut))
        hard = jnp.where(sel, jnp.float32(1.0), jnp.float32(0.0))
        out_ref[:, sl] = (hard - v) + v


def kernel(scores):
    bsz, nmax, _, ensemble = scores.shape
    rows = bsz * ensemble
    cols = nmax * nmax
    x = jnp.transpose(scores, (0, 3, 1, 2)).reshape(rows, cols)
    g = jax.random.gumbel(jax.random.key(42), x.shape, dtype=x.dtype)

    res = pl.pallas_call(
        _gumbel_topk_kernel,
        grid=(rows // _ROW_BLOCK,),
        in_specs=[
            pl.BlockSpec((_ROW_BLOCK, cols), lambda i: (i, 0)),
            pl.BlockSpec((_ROW_BLOCK, cols), lambda i: (i, 0)),
        ],
        out_specs=pl.BlockSpec((_ROW_BLOCK, cols), lambda i: (i, 0)),
        out_shape=jax.ShapeDtypeStruct((rows, cols), x.dtype),
        scratch_shapes=[
            pltpu.VMEM((_ROW_BLOCK, cols), jnp.float32),
            pltpu.VMEM((_ROW_BLOCK, cols), jnp.float32),
            pltpu.VMEM((_ROW_BLOCK, cols), jnp.float32),
        ],
        compiler_params=pltpu.CompilerParams(
            dimension_semantics=("parallel",),
        ),
    )(x, g)

    res = res.reshape(bsz, ensemble, nmax, nmax)
    return jnp.transpose(res, (0, 2, 3, 1))


# speedup vs baseline: 2.5653x; 1.0095x over previous
import jax
import jax.numpy as jnp
import numpy as np
from jax.experimental import pallas as pl
from jax.experimental.pallas import tpu as pltpu

_EPSILON = float(np.finfo(np.float32).tiny)
_K = 32
_TAU = 0.1

_ROW_BLOCK = 64
_CHUNK = 1024


def _gumbel_topk_kernel(x_ref, g_ref, out_ref, s_scr, khot_scr, e_scr):
    rows, cols = x_ref.shape
    ncheck = cols // _CHUNK
    chunks = [slice(c * _CHUNK, (c + 1) * _CHUNK) for c in range(ncheck)]
    inv_tau = jnp.float32(1.0) / jnp.float32(_TAU)

    def chunk_iota(c):
        it = jax.lax.broadcasted_iota(jnp.int32, (rows, _CHUNK), 1)
        return it + jnp.int32(c * _CHUNK)

    macc = None
    for sl in chunks:
        sv = x_ref[:, sl] + g_ref[:, sl]
        s_scr[:, sl] = sv
        khot_scr[:, sl] = jnp.abs(sv * 0.0)
        t = sv * inv_tau
        macc = t if macc is None else jnp.maximum(macc, t)
    m = jnp.max(macc, axis=1, keepdims=True)
    for sl in chunks:
        t = s_scr[:, sl] * inv_tau
        e_scr[:, sl] = jnp.exp(t - m)
    d = jnp.sum(e_scr[...], axis=1, keepdims=True)

    def soft_body(_, d):
        invd = 1.0 / d
        macc = None
        for sl in chunks:
            onehot = e_scr[:, sl] * invd
            khot_scr[:, sl] = khot_scr[:, sl] + onehot
            khot_mask = jnp.maximum(1.0 - onehot, _EPSILON)
            sv = s_scr[:, sl] + jnp.log(khot_mask)
            s_scr[:, sl] = sv
            t = sv * inv_tau
            macc = t if macc is None else jnp.maximum(macc, t)
        m = jnp.max(macc, axis=1, keepdims=True)
        for sl in chunks:
            t = s_scr[:, sl] * inv_tau
            e_scr[:, sl] = jnp.exp(t - m)
        return jnp.sum(e_scr[...], axis=1, keepdims=True)

    d = jax.lax.fori_loop(0, _K - 1, soft_body, d, unroll=False)

    invd = 1.0 / d
    for sl in chunks:
        khot_scr[:, sl] = khot_scr[:, sl] + e_scr[:, sl] * invd

    def khot_bits(sl):
        return jax.lax.bitcast_convert_type(khot_scr[:, sl], jnp.int32)

    def count_rows(acc):
        return jnp.sum(acc, axis=1, keepdims=True)

    lo0 = (d * 0.0).astype(jnp.int32)
    hi0 = lo0 + jnp.int32(0x7F800000)

    def vsearch_body(_, carry):
        lo, hi = carry
        mid = lo + jax.lax.shift_right_logical(hi - lo, 1)
        acc = None
        for sl in chunks:
            one = jnp.where(khot_bits(sl) >= mid, 1.0, 0.0)
            acc = one if acc is None else acc + one
        take = count_rows(acc) >= 32.0
        lo = jnp.where(take, mid, lo)
        hi = jnp.where(take, hi, mid)
        return lo, hi

    theta_b, _ = jax.lax.fori_loop(0, 31, vsearch_body, (lo0, hi0), unroll=False)
    theta = jax.lax.bitcast_convert_type(theta_b, jnp.float32)

    accg = None
    for sl in chunks:
        one = jnp.where(khot_bits(sl) >= theta_b + 1, 1.0, 0.0)
        accg = one if accg is None else accg + one
    r = 32.0 - count_rows(accg)

    ilo0 = lo0 - 1
    ihi0 = lo0 + jnp.int32(cols - 1)

    def isearch_body(_, carry):
        lo, hi = carry
        mid = lo + jax.lax.shift_right_logical(hi - lo, 1)
        acc = None
        for c, sl in enumerate(chunks):
            hit = (khot_bits(sl) == theta_b) & (chunk_iota(c) <= mid)
            one = jnp.where(hit, 1.0, 0.0)
            acc = one if acc is None else acc + one
        take = count_rows(acc) >= r
        lo = jnp.where(take, lo, mid)
        hi = jnp.where(take, mid, hi)
        return lo, hi

    _, idx_cut = jax.lax.fori_loop(0, 14, isearch_body, (ilo0, ihi0), unroll=False)

    for c, sl in enumerate(chunks):
        v = khot_scr[:, sl]
        it = chunk_iota(c)
        sel = (v > theta) | ((v == theta) & (it <= idx_cut))
        hard = jnp.where(sel, jnp.float32(1.0), jnp.float32(0.0))
        out_ref[:, sl] = (hard - v) + v


def kernel(scores):
    bsz, nmax, _, ensemble = scores.shape
    rows = bsz * ensemble
    cols = nmax * nmax
    x = jnp.transpose(scores, (0, 3, 1, 2)).reshape(rows, cols)
    g = jax.random.gumbel(jax.random.key(42), x.shape, dtype=x.dtype)

    res = pl.pallas_call(
        _gumbel_topk_kernel,
        grid=(rows // _ROW_BLOCK,),
        in_specs=[
            pl.BlockSpec((_ROW_BLOCK, cols), lambda i: (i, 0)),
            pl.BlockSpec((_ROW_BLOCK, cols), lambda i: (i, 0)),
        ],
        out_specs=pl.BlockSpec((_ROW_BLOCK, cols), lambda i: (i, 0)),
        out_shape=jax.ShapeDtypeStruct((rows, cols), x.dtype),
        scratch_shapes=[
            pltpu.VMEM((_ROW_BLOCK, cols), jnp.float32),
            pltpu.VMEM((_ROW_BLOCK, cols), jnp.float32),
            pltpu.VMEM((_ROW_BLOCK, cols), jnp.float32),
        ],
        compiler_params=pltpu.CompilerParams(
            dimension_semantics=("parallel",),
            allow_input_fusion=[True, True],
        ),
    )(x, g)

    res = res.reshape(bsz, ensemble, nmax, nmax)
    return jnp.transpose(res, (0, 2, 3, 1))
